# baseline (speedup 1.0000x reference)
import functools

import jax
import jax.numpy as jnp
from jax import lax
from jax.experimental import pallas as pl
from jax.experimental.pallas import tpu as pltpu
from jax.experimental.pallas import tpu_sc as plsc

N = 10000
NPAD = 10240
D = 128
H = 32
E = 320000
ROWS = 2560
EPAD = ROWS * 128
NC, NS = 2, 16
RPT = ROWS // (NC * NS)
DRT = ROWS // NS
STRIPE = NPAD // NS
YROWS = NPAD * H // 128
DISB_RPT = YROWS // (NC * NS)
NODES_PT = NPAD // (NC * NS)
PADIDX = N


def _sc_mesh():
    return plsc.VectorSubcoreMesh(
        core_axis_name="c", subcore_axis_name="s", num_cores=NC, num_subcores=NS
    )


def _sc_disb(ei2, z1d):
    @functools.partial(
        pl.kernel,
        out_type=jax.ShapeDtypeStruct((YROWS, 128), jnp.float32),
        mesh=_sc_mesh(),
        scratch_types=[
            pltpu.VMEM((DRT, 128), jnp.int32),
            pltpu.VMEM((NPAD,), jnp.float32),
            pltpu.VMEM((NS, NODES_PT), jnp.float32),
            pltpu.VMEM((NODES_PT,), jnp.float32),
            pltpu.VMEM((DISB_RPT, 128), jnp.float32),
            pltpu.VMEM_SHARED((NS, NPAD), jnp.float32),
        ],
        compiler_params=pltpu.CompilerParams(
            use_tc_tiling_on_sc=False, needs_layout_passes=False
        ),
    )
    def k(ei_hbm, z_hbm, disb_hbm, dst_v, hist_v, part_v, dis_v, disb_v, part_sh):
        cid = lax.axis_index("c")
        sid = lax.axis_index("s")
        pltpu.sync_copy(ei_hbm.at[1, pl.ds(sid * DRT, DRT)], dst_v)
        pltpu.sync_copy(z_hbm, hist_v)
        ones = jnp.ones((16,), jnp.float32)

        def hbody(j, carry):
            for kk in range(8):
                idx = dst_v[j, pl.ds(kk * 16, 16)]
                plsc.addupdate_scatter(hist_v, [idx], ones)
            return carry

        lax.fori_loop(0, DRT, hbody, 0)
        pltpu.sync_copy(hist_v, part_sh.at[sid])
        plsc.subcore_barrier()

        node0 = cid * (NPAD // NC) + sid * NODES_PT
        pltpu.sync_copy(part_sh.at[:, pl.ds(node0, NODES_PT)], part_v)

        def rbody(i, carry):
            s = jnp.ones((16,), jnp.float32)
            for t in range(NS):
                s = s + part_v[t, pl.ds(i * 16, 16)]
            bi = plsc.bitcast(s, jnp.int32)
            yi = plsc.bitcast(jnp.int32(0x5F3759DF) - (bi >> 1), jnp.float32)
            for _ in range(3):
                yi = yi * (1.5 - 0.5 * s * yi * yi)
            dis_v[pl.ds(i * 16, 16)] = yi
            return carry

        lax.fori_loop(0, NODES_PT // 16, rbody, 0)

        def ebody(g, carry):
            dvec = dis_v[pl.ds(g * 16, 16)]
            for l in range(16):
                v = dvec[l]
                row = g * 4 + l // 4
                col = (l % 4) * 32
                disb_v[row, pl.ds(col, 16)] = jnp.full((16,), v, jnp.float32)
                disb_v[row, pl.ds(col + 16, 16)] = jnp.full((16,), v, jnp.float32)
            return carry

        lax.fori_loop(0, NODES_PT // 16, ebody, 0)
        row0 = cid * (YROWS // NC) + sid * DISB_RPT
        pltpu.sync_copy(disb_v, disb_hbm.at[pl.ds(row0, DISB_RPT)])

    return k(ei2, z1d)


def _sc_aggregate(y, ei2, zrows):
    @functools.partial(
        pl.kernel,
        out_type=jax.ShapeDtypeStruct((NC, NPAD, H), jnp.float32),
        mesh=_sc_mesh(),
        scratch_types=[
            pltpu.VMEM((RPT, 128), jnp.int32),
            pltpu.VMEM((RPT, 128), jnp.int32),
            pltpu.VMEM((128, H), jnp.float32),
            pltpu.VMEM((128, H), jnp.float32),
            pltpu.VMEM((128, H), jnp.float32),
            pltpu.VMEM((128, H), jnp.float32),
            pltpu.VMEM((128, H), jnp.float32),
            pltpu.VMEM((128, H), jnp.float32),
            pltpu.VMEM((128, H), jnp.float32),
            pltpu.VMEM((128, H), jnp.float32),
            pltpu.VMEM((128, H), jnp.float32),
            pltpu.VMEM((128, H), jnp.float32),
            pltpu.VMEM((128, H), jnp.float32),
            pltpu.VMEM((128, H), jnp.float32),
            pltpu.VMEM((128, H), jnp.float32),
            pltpu.VMEM((128, H), jnp.float32),
            pltpu.VMEM((128, H), jnp.float32),
            pltpu.VMEM((128, H), jnp.float32),
            pltpu.SemaphoreType.DMA,
            pltpu.SemaphoreType.DMA,
            pltpu.SemaphoreType.DMA,
            pltpu.SemaphoreType.DMA,
            pltpu.SemaphoreType.DMA,
            pltpu.SemaphoreType.DMA,
            pltpu.SemaphoreType.DMA,
            pltpu.SemaphoreType.DMA,
            pltpu.SemaphoreType.DMA,
            pltpu.SemaphoreType.DMA,
            pltpu.SemaphoreType.DMA,
            pltpu.SemaphoreType.DMA,
            pltpu.SemaphoreType.DMA,
            pltpu.SemaphoreType.DMA,
            pltpu.SemaphoreType.DMA,
            pltpu.SemaphoreType.DMA,
            pltpu.VMEM_SHARED((NPAD, H), jnp.float32),
        ],
        compiler_params=pltpu.CompilerParams(use_tc_tiling_on_sc=False),
    )
    def k(y_hbm, ei_hbm, z_hbm, out_hbm,
          src_v, dst_v, *bufsems):
        acc_sh = bufsems[-1]
        bufs = tuple(bufsems[0:16])
        sems = tuple(bufsems[16:32])
        cid = lax.axis_index("c")
        sid = lax.axis_index("s")
        row0 = cid * (ROWS // NC) + sid * RPT
        pltpu.sync_copy(ei_hbm.at[0, pl.ds(row0, RPT)], src_v)
        pltpu.sync_copy(ei_hbm.at[1, pl.ds(row0, RPT)], dst_v)
        pltpu.sync_copy(z_hbm.at[pl.ds(sid * STRIPE, STRIPE)],
                        acc_sh.at[pl.ds(sid * STRIPE, STRIPE)])
        plsc.subcore_barrier()

        NBUF = 16
        for j0 in range(NBUF - 1):
            pltpu.async_copy(y_hbm.at[src_v.at[j0]], bufs[j0], sems[j0])

        def body(i, carry):
            for b in range(NBUF):
                j = NBUF * i + b
                jn = j + NBUF - 1
                bn = (b + NBUF - 1) % NBUF

                @pl.when(jn < RPT)
                def _():
                    pltpu.async_copy(y_hbm.at[src_v.at[jn]], bufs[bn], sems[bn])

                pltpu.make_async_copy(y_hbm.at[src_v.at[j]], bufs[b], sems[b]).wait()
                pltpu.sync_copy(bufs[b], acc_sh.at[dst_v.at[j]], add=True)
            return carry

        lax.fori_loop(0, RPT // NBUF, body, 0)
        plsc.subcore_barrier()
        pltpu.sync_copy(acc_sh.at[pl.ds(sid * STRIPE, STRIPE)],
                        out_hbm.at[cid, pl.ds(sid * STRIPE, STRIPE)])

    return k(y, ei2, zrows)


def _tc_first(x4, w1b, disb):
    def body(x_ref, w_ref, disb_ref, y_ref):
        xw = jnp.dot(x_ref[...], w_ref[...], preferred_element_type=jnp.float32)
        y_ref[...] = disb_ref[...] * xw

    return pl.pallas_call(
        body,
        out_shape=jax.ShapeDtypeStruct((YROWS, 128), jnp.float32),
    )(x4, w1b, disb)


def _tc_mid(p, y1, disb, b1t, w2b):
    def body(p_ref, y1_ref, disb_ref, b_ref, w_ref, y2_ref):
        disb = disb_ref[...]
        agg = p_ref[pl.ds(0, YROWS), :] + p_ref[pl.ds(YROWS, YROWS), :] + y1_ref[...]
        h = jnp.maximum(disb * agg + b_ref[...], 0.0)
        y2_ref[...] = disb * jnp.dot(h, w_ref[...], preferred_element_type=jnp.float32)

    return pl.pallas_call(
        body,
        out_shape=jax.ShapeDtypeStruct((YROWS, 128), jnp.float32),
    )(p, y1, disb, b1t, w2b)


def _tc_final(p, y2, disb, b2t, wfb, bft):
    def body(p_ref, y2_ref, disb_ref, b_ref, wf_ref, bf_ref, s_ref):
        agg = p_ref[pl.ds(0, YROWS), :] + p_ref[pl.ds(YROWS, YROWS), :] + y2_ref[...]
        h = jnp.maximum(disb_ref[...] * agg + b_ref[...], 0.0)
        s_ref[...] = jnp.dot(h, wf_ref[...], preferred_element_type=jnp.float32) + bf_ref[...]

    return pl.pallas_call(
        body,
        out_shape=jax.ShapeDtypeStruct((YROWS, 4), jnp.float32),
    )(p, y2, disb, b2t, wfb, bft)


def kernel(x, edge_index, W1, b1, W2, b2, Wf, bf):
    pad = PADIDX + jnp.arange(EPAD - E, dtype=jnp.int32) % (NPAD - N)
    pad2 = jnp.broadcast_to(pad, (2, EPAD - E))
    ei2 = jnp.concatenate([edge_index.astype(jnp.int32), pad2], axis=1)
    ei2 = ei2.reshape(2, ROWS, 128)

    x4 = jnp.zeros((YROWS, 4 * D), jnp.float32).at[: N // 4].set(
        x.reshape(N // 4, 4 * D))
    z1d = jnp.zeros((NPAD,), jnp.float32)
    zrows = jnp.zeros((NPAD, H), jnp.float32)

    blk = jax.scipy.linalg.block_diag
    w1b = blk(W1, W1, W1, W1)
    w2b = blk(W2, W2, W2, W2)
    wfb = blk(Wf, Wf, Wf, Wf)
    b1t = jnp.tile(b1, 4).reshape(1, 128)
    b2t = jnp.tile(b2, 4).reshape(1, 128)
    bft = jnp.broadcast_to(bf.reshape(1, 1), (1, 4))

    disb = _sc_disb(ei2, z1d)
    y1 = _tc_first(x4, w1b, disb)
    p1 = _sc_aggregate(y1.reshape(NPAD, H), ei2, zrows)
    y2 = _tc_mid(p1.reshape(2 * YROWS, 128), y1, disb, b1t, w2b)
    p2 = _sc_aggregate(y2.reshape(NPAD, H), ei2, zrows)
    s4 = _tc_final(p2.reshape(2 * YROWS, 128), y2, disb, b2t, wfb, bft)
    return s4.reshape(NPAD)[:N]


# speedup vs baseline: 1.0181x; 1.0181x over previous
import functools

import jax
import jax.numpy as jnp
from jax import lax
from jax.experimental import pallas as pl
from jax.experimental.pallas import tpu as pltpu
from jax.experimental.pallas import tpu_sc as plsc

N = 10000
NPAD = 10240
D = 128
H = 32
E = 320000
ROWS = 2560
EPAD = ROWS * 128
NC, NS = 2, 16
RPT = ROWS // (NC * NS)
DRT = ROWS // NS
STRIPE = NPAD // NS
YROWS = NPAD * H // 128
DISB_RPT = YROWS // (NC * NS)
NODES_PT = NPAD // (NC * NS)
PADIDX = N


def _sc_mesh():
    return plsc.VectorSubcoreMesh(
        core_axis_name="c", subcore_axis_name="s", num_cores=NC, num_subcores=NS
    )


def _sc_disb(ei2, z1d):
    @functools.partial(
        pl.kernel,
        out_type=jax.ShapeDtypeStruct((YROWS, 128), jnp.float32),
        mesh=_sc_mesh(),
        scratch_types=[
            pltpu.VMEM((DRT, 128), jnp.int32),
            pltpu.VMEM((NPAD,), jnp.float32),
            pltpu.VMEM((NS, NODES_PT), jnp.float32),
            pltpu.VMEM((NODES_PT,), jnp.float32),
            pltpu.VMEM((DISB_RPT, 128), jnp.float32),
            pltpu.VMEM_SHARED((NS, NPAD), jnp.float32),
        ],
        compiler_params=pltpu.CompilerParams(
            use_tc_tiling_on_sc=False, needs_layout_passes=False
        ),
    )
    def k(ei_hbm, z_hbm, disb_hbm, dst_v, hist_v, part_v, dis_v, disb_v, part_sh):
        cid = lax.axis_index("c")
        sid = lax.axis_index("s")
        pltpu.sync_copy(ei_hbm.at[1, pl.ds(sid * DRT, DRT)], dst_v)
        pltpu.sync_copy(z_hbm, hist_v)
        ones = jnp.ones((16,), jnp.float32)

        def hbody(j, carry):
            for kk in range(8):
                idx = dst_v[j, pl.ds(kk * 16, 16)]
                plsc.addupdate_scatter(hist_v, [idx], ones)
            return carry

        lax.fori_loop(0, DRT, hbody, 0)
        pltpu.sync_copy(hist_v, part_sh.at[sid])
        plsc.subcore_barrier()

        node0 = cid * (NPAD // NC) + sid * NODES_PT
        pltpu.sync_copy(part_sh.at[:, pl.ds(node0, NODES_PT)], part_v)

        def rbody(i, carry):
            s = jnp.ones((16,), jnp.float32)
            for t in range(NS):
                s = s + part_v[t, pl.ds(i * 16, 16)]
            bi = plsc.bitcast(s, jnp.int32)
            yi = plsc.bitcast(jnp.int32(0x5F3759DF) - (bi >> 1), jnp.float32)
            for _ in range(3):
                yi = yi * (1.5 - 0.5 * s * yi * yi)
            dis_v[pl.ds(i * 16, 16)] = yi
            return carry

        lax.fori_loop(0, NODES_PT // 16, rbody, 0)

        def ebody(g, carry):
            dvec = dis_v[pl.ds(g * 16, 16)]
            for l in range(16):
                v = dvec[l]
                row = g * 4 + l // 4
                col = (l % 4) * 32
                disb_v[row, pl.ds(col, 16)] = jnp.full((16,), v, jnp.float32)
                disb_v[row, pl.ds(col + 16, 16)] = jnp.full((16,), v, jnp.float32)
            return carry

        lax.fori_loop(0, NODES_PT // 16, ebody, 0)
        row0 = cid * (YROWS // NC) + sid * DISB_RPT
        pltpu.sync_copy(disb_v, disb_hbm.at[pl.ds(row0, DISB_RPT)])

    return k(ei2, z1d)


def _sc_aggregate(y, ei2, zrows):
    @functools.partial(
        pl.kernel,
        out_type=jax.ShapeDtypeStruct((NC, NPAD, H), jnp.float32),
        mesh=_sc_mesh(),
        scratch_types=[
            pltpu.VMEM((RPT, 128), jnp.int32),
            pltpu.VMEM((RPT, 128), jnp.int32),
            pltpu.VMEM((128, H), jnp.float32),
            pltpu.VMEM((128, H), jnp.float32),
            pltpu.VMEM((128, H), jnp.float32),
            pltpu.VMEM((128, H), jnp.float32),
            pltpu.VMEM((128, H), jnp.float32),
            pltpu.VMEM((128, H), jnp.float32),
            pltpu.VMEM((128, H), jnp.float32),
            pltpu.VMEM((128, H), jnp.float32),
            pltpu.SemaphoreType.DMA,
            pltpu.SemaphoreType.DMA,
            pltpu.SemaphoreType.DMA,
            pltpu.SemaphoreType.DMA,
            pltpu.SemaphoreType.DMA,
            pltpu.SemaphoreType.DMA,
            pltpu.SemaphoreType.DMA,
            pltpu.SemaphoreType.DMA,
            pltpu.VMEM_SHARED((NPAD, H), jnp.float32),
        ],
        compiler_params=pltpu.CompilerParams(use_tc_tiling_on_sc=False),
    )
    def k(y_hbm, ei_hbm, z_hbm, out_hbm,
          src_v, dst_v, *bufsems):
        acc_sh = bufsems[-1]
        bufs = tuple(bufsems[0:8])
        sems = tuple(bufsems[8:16])
        cid = lax.axis_index("c")
        sid = lax.axis_index("s")
        row0 = cid * (ROWS // NC) + sid * RPT
        pltpu.sync_copy(ei_hbm.at[0, pl.ds(row0, RPT)], src_v)
        pltpu.sync_copy(ei_hbm.at[1, pl.ds(row0, RPT)], dst_v)
        pltpu.sync_copy(z_hbm.at[pl.ds(sid * STRIPE, STRIPE)],
                        acc_sh.at[pl.ds(sid * STRIPE, STRIPE)])
        plsc.subcore_barrier()

        NBUF = 8
        for j0 in range(NBUF - 1):
            pltpu.async_copy(y_hbm.at[src_v.at[j0]], bufs[j0], sems[j0])

        def body(i, carry):
            for b in range(NBUF):
                j = NBUF * i + b
                jn = j + NBUF - 1
                bn = (b + NBUF - 1) % NBUF

                @pl.when(jn < RPT)
                def _():
                    pltpu.async_copy(y_hbm.at[src_v.at[jn]], bufs[bn], sems[bn])

                pltpu.make_async_copy(y_hbm.at[src_v.at[j]], bufs[b], sems[b]).wait()
                pltpu.sync_copy(bufs[b], acc_sh.at[dst_v.at[j]], add=True)
            return carry

        lax.fori_loop(0, RPT // NBUF, body, 0)
        plsc.subcore_barrier()
        pltpu.sync_copy(acc_sh.at[pl.ds(sid * STRIPE, STRIPE)],
                        out_hbm.at[cid, pl.ds(sid * STRIPE, STRIPE)])

    return k(y, ei2, zrows)


def _tc_first(x4, w1b, disb):
    def body(x_ref, w_ref, disb_ref, y_ref):
        xw = jnp.dot(x_ref[...], w_ref[...], preferred_element_type=jnp.float32)
        y_ref[...] = disb_ref[...] * xw

    return pl.pallas_call(
        body,
        out_shape=jax.ShapeDtypeStruct((YROWS, 128), jnp.float32),
    )(x4, w1b, disb)


def _tc_mid(p, y1, disb, b1t, w2b):
    def body(p_ref, y1_ref, disb_ref, b_ref, w_ref, y2_ref):
        disb = disb_ref[...]
        agg = p_ref[pl.ds(0, YROWS), :] + p_ref[pl.ds(YROWS, YROWS), :] + y1_ref[...]
        h = jnp.maximum(disb * agg + b_ref[...], 0.0)
        y2_ref[...] = disb * jnp.dot(h, w_ref[...], preferred_element_type=jnp.float32)

    return pl.pallas_call(
        body,
        out_shape=jax.ShapeDtypeStruct((YROWS, 128), jnp.float32),
    )(p, y1, disb, b1t, w2b)


def _tc_final(p, y2, disb, b2t, wfb, bft):
    def body(p_ref, y2_ref, disb_ref, b_ref, wf_ref, bf_ref, s_ref):
        agg = p_ref[pl.ds(0, YROWS), :] + p_ref[pl.ds(YROWS, YROWS), :] + y2_ref[...]
        h = jnp.maximum(disb_ref[...] * agg + b_ref[...], 0.0)
        s_ref[...] = jnp.dot(h, wf_ref[...], preferred_element_type=jnp.float32) + bf_ref[...]

    return pl.pallas_call(
        body,
        out_shape=jax.ShapeDtypeStruct((YROWS, 4), jnp.float32),
    )(p, y2, disb, b2t, wfb, bft)


def kernel(x, edge_index, W1, b1, W2, b2, Wf, bf):
    pad = PADIDX + jnp.arange(EPAD - E, dtype=jnp.int32) % (NPAD - N)
    pad2 = jnp.broadcast_to(pad, (2, EPAD - E))
    ei2 = jnp.concatenate([edge_index.astype(jnp.int32), pad2], axis=1)
    ei2 = ei2.reshape(2, ROWS, 128)

    x4 = jnp.zeros((YROWS, 4 * D), jnp.float32).at[: N // 4].set(
        x.reshape(N // 4, 4 * D))
    z1d = jnp.zeros((NPAD,), jnp.float32)
    zrows = jnp.zeros((NPAD, H), jnp.float32)

    blk = jax.scipy.linalg.block_diag
    w1b = blk(W1, W1, W1, W1)
    w2b = blk(W2, W2, W2, W2)
    wfb = blk(Wf, Wf, Wf, Wf)
    b1t = jnp.tile(b1, 4).reshape(1, 128)
    b2t = jnp.tile(b2, 4).reshape(1, 128)
    bft = jnp.broadcast_to(bf.reshape(1, 1), (1, 4))

    disb = _sc_disb(ei2, z1d)
    y1 = _tc_first(x4, w1b, disb)
    p1 = _sc_aggregate(y1.reshape(NPAD, H), ei2, zrows)
    y2 = _tc_mid(p1.reshape(2 * YROWS, 128), y1, disb, b1t, w2b)
    p2 = _sc_aggregate(y2.reshape(NPAD, H), ei2, zrows)
    s4 = _tc_final(p2.reshape(2 * YROWS, 128), y2, disb, b2t, wfb, bft)
    return s4.reshape(NPAD)[:N]
